# baseline (speedup 1.0000x reference)
import jax
import jax.numpy as jnp
from jax.experimental import pallas as pl

B = 8; NPC = 2048; D = 6; CIN = 128; COUT = 128; CDELTA = 32; K = 16
RATIO = 0.5; RAD = 2.0; MAXN = 64
M = int(NPC * RATIO)
CMID = CIN + CDELTA
T = B * M


def _fps_cloud(pos_c):
    def body(i, state):
        idx, mind = state
        last = pos_c[idx[i - 1]]
        d = jnp.sum((pos_c - last) ** 2, axis=-1)
        mind = jnp.minimum(mind, d)
        nxt = jnp.argmax(mind).astype(jnp.int32)
        return idx.at[i].set(nxt), mind
    idx0 = jnp.zeros((M,), jnp.int32)
    mind0 = jnp.full((NPC,), jnp.inf, dtype=jnp.float32)
    idx, _ = jax.lax.fori_loop(1, M, body, (idx0, mind0))
    return idx


def _grouped_conv1d(t, w, b):
    wr = w.reshape(K, K, K)
    return jnp.einsum('tik,imk->tim', t, wr).reshape(t.shape[0], K * K) + b


def _final_linear_kernel(feat_ref, wl_ref, bl_ref, out_ref):
    out_ref[...] = (
        jnp.dot(feat_ref[...], wl_ref[...], preferred_element_type=jnp.float32)
        + bl_ref[...]
    )


def kernel(x, pos, batch, w1, b1, w2, b2, w3, b3, cw1, cb1, cw2, cb2, dw, db, wl, bl):
    posr = pos.reshape(B, NPC, D)
    xr = x.reshape(B, NPC, CIN)
    idx = jax.vmap(_fps_cloud)(jax.lax.stop_gradient(posr))
    pos_s = jax.vmap(lambda a, i: a[i])(posr, idx)
    d2 = (jnp.sum(pos_s ** 2, -1)[:, :, None] + jnp.sum(posr ** 2, -1)[:, None, :]
          - 2.0 * jnp.einsum('bmd,bnd->bmn', pos_s, posr))
    negd, nbr = jax.lax.top_k(-jax.lax.stop_gradient(d2), MAXN)
    valid = (-negd) <= RAD * RAD
    nbr = jnp.where(valid, nbr, nbr[..., :1])
    nbr16 = nbr[..., :K]
    pos_n = jax.vmap(lambda a, i: a[i])(posr, nbr16)
    x_n = jax.vmap(lambda a, i: a[i])(xr, nbr16)
    pos_rel = pos_n - pos_s[:, :, None, :]
    h = jax.nn.elu(pos_rel.reshape(T * K, D) @ w1 + b1)
    h = jax.nn.elu(h @ w2 + b2)
    delta = h.reshape(T, K, CDELTA)
    xg = x_n.reshape(T, K, CIN)
    x_star = jnp.concatenate([delta, xg], axis=-1)
    x_star = jnp.transpose(x_star, (0, 2, 1))
    t = jax.nn.elu(pos_rel.reshape(T, K * D) @ w3 + b3)
    t = t.reshape(T, K, K)
    t = jax.nn.elu(_grouped_conv1d(t, cw1, cb1)).reshape(T, K, K)
    trans = _grouped_conv1d(t, cw2, cb2).reshape(T, K, K)
    x_t = jnp.matmul(x_star, trans)
    feat = jnp.einsum('tck,ck->tc', x_t, dw) + db
    out = pl.pallas_call(
        _final_linear_kernel,
        out_shape=jax.ShapeDtypeStruct((T, COUT), jnp.float32),
    )(feat, wl, bl)
    gidx = (idx.astype(jnp.int64) + (jnp.arange(B, dtype=jnp.int64) * NPC)[:, None]).reshape(-1)
    pos_out = pos[gidx]
    batch_out = batch[gidx]
    return out, pos_out, batch_out


# speedup vs baseline: 1.9993x; 1.9993x over previous
import jax
import jax.numpy as jnp
from jax.experimental import pallas as pl

B = 8; NPC = 2048; D = 6; CIN = 128; COUT = 128; CDELTA = 32; K = 16
RATIO = 0.5; RAD = 2.0; MAXN = 64
M = int(NPC * RATIO)
CMID = CIN + CDELTA
T = B * M


def _fps_kernel(pos6_ref, idx_ref):
    col = jax.lax.broadcasted_iota(jnp.int32, (B, NPC), 1)
    colm = jax.lax.broadcasted_iota(jnp.int32, (B, M), 1)
    p = [pos6_ref[d] for d in range(D)]

    def body(i, carry):
        sel, mind, idxacc = carry
        maskc = col == sel
        a = []
        for d in range(D):
            lastd = jnp.sum(jnp.where(maskc, p[d], 0.0), axis=1, keepdims=True)
            ad = p[d] - lastd
            a.append(ad * ad)
        dsum = ((a[0] + a[4]) + a[2]) + ((a[1] + a[5]) + a[3])
        mind = jnp.minimum(mind, dsum)
        mx = jnp.max(mind, axis=1, keepdims=True)
        cand = jnp.where(mind == mx, col, NPC)
        sel = jnp.min(cand, axis=1, keepdims=True)
        idxacc = jnp.where(colm == i, sel, idxacc)
        return sel, mind, idxacc

    sel0 = jnp.zeros((B, 1), jnp.int32)
    mind0 = jnp.full((B, NPC), jnp.inf, dtype=jnp.float32)
    idx0 = jnp.zeros((B, M), jnp.int32)
    _, _, idxacc = jax.lax.fori_loop(1, M, body, (sel0, mind0, idx0))
    idx_ref[...] = idxacc


def _fps_pallas(posr):
    pos6 = jnp.transpose(posr, (2, 0, 1))
    return pl.pallas_call(
        _fps_kernel,
        out_shape=jax.ShapeDtypeStruct((B, M), jnp.int32),
    )(pos6)


def _grouped_conv1d(t, w, b):
    wr = w.reshape(K, K, K)
    return jnp.einsum('tik,imk->tim', t, wr).reshape(t.shape[0], K * K) + b


def _final_linear_kernel(feat_ref, wl_ref, bl_ref, out_ref):
    out_ref[...] = (
        jnp.dot(feat_ref[...], wl_ref[...], preferred_element_type=jnp.float32)
        + bl_ref[...]
    )


def kernel(x, pos, batch, w1, b1, w2, b2, w3, b3, cw1, cb1, cw2, cb2, dw, db, wl, bl):
    posr = pos.reshape(B, NPC, D)
    xr = x.reshape(B, NPC, CIN)
    idx = _fps_pallas(posr)
    pos_s = jax.vmap(lambda a, i: a[i])(posr, idx)
    d2 = (jnp.sum(pos_s ** 2, -1)[:, :, None] + jnp.sum(posr ** 2, -1)[:, None, :]
          - 2.0 * jnp.einsum('bmd,bnd->bmn', pos_s, posr))
    negd, nbr = jax.lax.top_k(-jax.lax.stop_gradient(d2), MAXN)
    valid = (-negd) <= RAD * RAD
    nbr = jnp.where(valid, nbr, nbr[..., :1])
    nbr16 = nbr[..., :K]
    pos_n = jax.vmap(lambda a, i: a[i])(posr, nbr16)
    x_n = jax.vmap(lambda a, i: a[i])(xr, nbr16)
    pos_rel = pos_n - pos_s[:, :, None, :]
    h = jax.nn.elu(pos_rel.reshape(T * K, D) @ w1 + b1)
    h = jax.nn.elu(h @ w2 + b2)
    delta = h.reshape(T, K, CDELTA)
    xg = x_n.reshape(T, K, CIN)
    x_star = jnp.concatenate([delta, xg], axis=-1)
    x_star = jnp.transpose(x_star, (0, 2, 1))
    t = jax.nn.elu(pos_rel.reshape(T, K * D) @ w3 + b3)
    t = t.reshape(T, K, K)
    t = jax.nn.elu(_grouped_conv1d(t, cw1, cb1)).reshape(T, K, K)
    trans = _grouped_conv1d(t, cw2, cb2).reshape(T, K, K)
    x_t = jnp.matmul(x_star, trans)
    feat = jnp.einsum('tck,ck->tc', x_t, dw) + db
    out = pl.pallas_call(
        _final_linear_kernel,
        out_shape=jax.ShapeDtypeStruct((T, COUT), jnp.float32),
    )(feat, wl, bl)
    gidx = (idx.astype(jnp.int64) + (jnp.arange(B, dtype=jnp.int64) * NPC)[:, None]).reshape(-1)
    pos_out = pos[gidx]
    batch_out = batch[gidx]
    return out, pos_out, batch_out


# speedup vs baseline: 2.9425x; 1.4718x over previous
import jax
import jax.numpy as jnp
from jax.experimental import pallas as pl

B = 8; NPC = 2048; D = 6; CIN = 128; COUT = 128; CDELTA = 32; K = 16
RATIO = 0.5; RAD = 2.0; MAXN = 64
M = int(NPC * RATIO)
CMID = CIN + CDELTA
T = B * M


def _fps_kernel(pos6_ref, idx_ref):
    col = jax.lax.broadcasted_iota(jnp.int32, (B, NPC), 1)
    colm = jax.lax.broadcasted_iota(jnp.int32, (B, M), 1)
    p = [pos6_ref[d] for d in range(D)]

    def body(i, carry):
        sel, mind, idxacc = carry
        maskc = col == sel
        a = []
        for d in range(D):
            lastd = jnp.sum(jnp.where(maskc, p[d], 0.0), axis=1, keepdims=True)
            ad = p[d] - lastd
            a.append(ad * ad)
        dsum = ((a[0] + a[4]) + a[2]) + ((a[1] + a[5]) + a[3])
        mind = jnp.minimum(mind, dsum)
        mx = jnp.max(mind, axis=1, keepdims=True)
        cand = jnp.where(mind == mx, col, NPC)
        sel = jnp.min(cand, axis=1, keepdims=True)
        idxacc = jnp.where(colm == i, sel, idxacc)
        return sel, mind, idxacc

    sel0 = jnp.zeros((B, 1), jnp.int32)
    mind0 = jnp.full((B, NPC), jnp.inf, dtype=jnp.float32)
    idx0 = jnp.zeros((B, M), jnp.int32)
    _, _, idxacc = jax.lax.fori_loop(1, M, body, (sel0, mind0, idx0))
    idx_ref[...] = idxacc


def _fps_pallas(posr):
    pos6 = jnp.transpose(posr, (2, 0, 1))
    return pl.pallas_call(
        _fps_kernel,
        out_shape=jax.ShapeDtypeStruct((B, M), jnp.int32),
    )(pos6)


QB = 128


def _nbr_kernel(pos6p_ref, posp_ref, idxq_ref, nbr_ref, poss_ref):
    pc = pos6p_ref[0]
    idxq = idxq_ref[0, 0]
    col = jax.lax.broadcasted_iota(jnp.int32, (QB, NPC), 1)
    sel_mask = col == idxq
    lane8 = jax.lax.broadcasted_iota(jnp.int32, (1, 8), 1)
    pos_s = jnp.zeros((QB, 8), jnp.float32)
    ps_cols = []
    for d in range(D):
        psd = jnp.sum(jnp.where(sel_mask, pc[d:d + 1, :], 0.0),
                      axis=1, keepdims=True)
        ps_cols.append(psd)
        pos_s = pos_s + psd * (lane8 == d).astype(jnp.float32)
    a = [ps_cols[d] * ps_cols[d] for d in range(D)]
    A = ((a[0] + a[4]) + a[2]) + ((a[1] + a[5]) + a[3])
    bsq = [pc[d:d + 1, :] * pc[d:d + 1, :] for d in range(D)]
    Bv = ((bsq[0] + bsq[4]) + bsq[2]) + ((bsq[1] + bsq[5]) + bsq[3])
    pn = posp_ref[0]
    C = jax.lax.dot_general(pos_s, pn, (((1,), (1,)), ((), ())),
                            preferred_element_type=jnp.float32)
    d2 = (A + Bv) - 2.0 * C
    work = d2
    first = None
    for s in range(K):
        mn = jnp.min(work, axis=1, keepdims=True)
        cand = jnp.where(work == mn, col, NPC)
        sidx = jnp.min(cand, axis=1, keepdims=True)
        if s == 0:
            first = sidx
            sel = sidx
        else:
            sel = jnp.where(mn <= RAD * RAD, sidx, first)
        nbr_ref[0, 0, :, s:s + 1] = sel
        work = jnp.where(col == sidx, jnp.inf, work)
    poss_ref[0, 0] = pos_s


def _nbr_pallas(posr, idx):
    pos6p = jnp.pad(jnp.transpose(posr, (0, 2, 1)), ((0, 0), (0, 2), (0, 0)))
    posp = jnp.pad(posr, ((0, 0), (0, 0), (0, 2)))
    idx4 = idx.reshape(B, M // QB, QB, 1)
    nbr, poss = pl.pallas_call(
        _nbr_kernel,
        grid=(B, M // QB),
        in_specs=[
            pl.BlockSpec((1, 8, NPC), lambda b, q: (b, 0, 0)),
            pl.BlockSpec((1, NPC, 8), lambda b, q: (b, 0, 0)),
            pl.BlockSpec((1, 1, QB, 1), lambda b, q: (b, q, 0, 0)),
        ],
        out_specs=[
            pl.BlockSpec((1, 1, QB, K), lambda b, q: (b, q, 0, 0)),
            pl.BlockSpec((1, 1, QB, 8), lambda b, q: (b, q, 0, 0)),
        ],
        out_shape=[
            jax.ShapeDtypeStruct((B, M // QB, QB, K), jnp.int32),
            jax.ShapeDtypeStruct((B, M // QB, QB, 8), jnp.float32),
        ],
    )(pos6p, posp, idx4)
    return nbr.reshape(B, M, K), poss.reshape(B, M, 8)


def _grouped_conv1d(t, w, b):
    wr = w.reshape(K, K, K)
    return jnp.einsum('tik,imk->tim', t, wr).reshape(t.shape[0], K * K) + b


def _final_linear_kernel(feat_ref, wl_ref, bl_ref, out_ref):
    out_ref[...] = (
        jnp.dot(feat_ref[...], wl_ref[...], preferred_element_type=jnp.float32)
        + bl_ref[...]
    )


def kernel(x, pos, batch, w1, b1, w2, b2, w3, b3, cw1, cb1, cw2, cb2, dw, db, wl, bl):
    posr = pos.reshape(B, NPC, D)
    xr = x.reshape(B, NPC, CIN)
    idx = _fps_pallas(posr)
    nbr16, poss8 = _nbr_pallas(posr, idx)
    pos_s = poss8[..., :D]
    pos_n = jax.vmap(lambda a, i: a[i])(posr, nbr16)
    x_n = jax.vmap(lambda a, i: a[i])(xr, nbr16)
    pos_rel = pos_n - pos_s[:, :, None, :]
    h = jax.nn.elu(pos_rel.reshape(T * K, D) @ w1 + b1)
    h = jax.nn.elu(h @ w2 + b2)
    delta = h.reshape(T, K, CDELTA)
    xg = x_n.reshape(T, K, CIN)
    x_star = jnp.concatenate([delta, xg], axis=-1)
    x_star = jnp.transpose(x_star, (0, 2, 1))
    t = jax.nn.elu(pos_rel.reshape(T, K * D) @ w3 + b3)
    t = t.reshape(T, K, K)
    t = jax.nn.elu(_grouped_conv1d(t, cw1, cb1)).reshape(T, K, K)
    trans = _grouped_conv1d(t, cw2, cb2).reshape(T, K, K)
    x_t = jnp.matmul(x_star, trans)
    feat = jnp.einsum('tck,ck->tc', x_t, dw) + db
    out = pl.pallas_call(
        _final_linear_kernel,
        out_shape=jax.ShapeDtypeStruct((T, COUT), jnp.float32),
    )(feat, wl, bl)
    gidx = (idx.astype(jnp.int64) + (jnp.arange(B, dtype=jnp.int64) * NPC)[:, None]).reshape(-1)
    pos_out = pos_s.reshape(T, D)
    batch_out = batch[gidx]
    return out, pos_out, batch_out


# speedup vs baseline: 11.6419x; 3.9564x over previous
import functools

import jax
import jax.numpy as jnp
from jax import lax
from jax.experimental import pallas as pl
from jax.experimental.pallas import tpu as pltpu
from jax.experimental.pallas import tpu_sc as plsc

B = 8; NPC = 2048; D = 6; CIN = 128; COUT = 128; CDELTA = 32; K = 16
RATIO = 0.5; RAD = 2.0; MAXN = 64
M = int(NPC * RATIO)
CMID = CIN + CDELTA
T = B * M


def _fps_kernel(pos6_ref, idx_ref):
    col = jax.lax.broadcasted_iota(jnp.int32, (B, NPC), 1)
    colm = jax.lax.broadcasted_iota(jnp.int32, (B, M), 1)
    p = [pos6_ref[d] for d in range(D)]

    def body(i, carry):
        sel, mind, idxacc = carry
        maskc = col == sel
        a = []
        for d in range(D):
            lastd = jnp.sum(jnp.where(maskc, p[d], 0.0), axis=1, keepdims=True)
            ad = p[d] - lastd
            a.append(ad * ad)
        dsum = ((a[0] + a[4]) + a[2]) + ((a[1] + a[5]) + a[3])
        mind = jnp.minimum(mind, dsum)
        mx = jnp.max(mind, axis=1, keepdims=True)
        cand = jnp.where(mind == mx, col, NPC)
        sel = jnp.min(cand, axis=1, keepdims=True)
        idxacc = jnp.where(colm == i, sel, idxacc)
        return sel, mind, idxacc

    sel0 = jnp.zeros((B, 1), jnp.int32)
    mind0 = jnp.full((B, NPC), jnp.inf, dtype=jnp.float32)
    idx0 = jnp.zeros((B, M), jnp.int32)
    _, _, idxacc = jax.lax.fori_loop(1, M, body, (sel0, mind0, idx0))
    idx_ref[...] = idxacc


def _fps_pallas(posr):
    pos6 = jnp.transpose(posr, (2, 0, 1))
    return pl.pallas_call(
        _fps_kernel,
        out_shape=jax.ShapeDtypeStruct((B, M), jnp.int32),
    )(pos6)


QB = 128


def _nbr_kernel(pos6p_ref, posp_ref, idxq_ref, nbr_ref, poss_ref):
    pc = pos6p_ref[0]
    idxq = idxq_ref[0, 0]
    col = jax.lax.broadcasted_iota(jnp.int32, (QB, NPC), 1)
    sel_mask = col == idxq
    lane8 = jax.lax.broadcasted_iota(jnp.int32, (1, 8), 1)
    pos_s = jnp.zeros((QB, 8), jnp.float32)
    ps_cols = []
    for d in range(D):
        psd = jnp.sum(jnp.where(sel_mask, pc[d:d + 1, :], 0.0),
                      axis=1, keepdims=True)
        ps_cols.append(psd)
        pos_s = pos_s + psd * (lane8 == d).astype(jnp.float32)
    a = [ps_cols[d] * ps_cols[d] for d in range(D)]
    A = ((a[0] + a[4]) + a[2]) + ((a[1] + a[5]) + a[3])
    bsq = [pc[d:d + 1, :] * pc[d:d + 1, :] for d in range(D)]
    Bv = ((bsq[0] + bsq[4]) + bsq[2]) + ((bsq[1] + bsq[5]) + bsq[3])
    pn = posp_ref[0]
    C = jax.lax.dot_general(pos_s, pn, (((1,), (1,)), ((), ())),
                            preferred_element_type=jnp.float32)
    d2 = (A + Bv) - 2.0 * C
    work = d2
    first = None
    for s in range(K):
        mn = jnp.min(work, axis=1, keepdims=True)
        cand = jnp.where(work == mn, col, NPC)
        sidx = jnp.min(cand, axis=1, keepdims=True)
        if s == 0:
            first = sidx
            sel = sidx
        else:
            sel = jnp.where(mn <= RAD * RAD, sidx, first)
        nbr_ref[0, 0, :, s:s + 1] = sel
        work = jnp.where(col == sidx, jnp.inf, work)
    poss_ref[0, 0] = pos_s


def _nbr_pallas(posr, idx):
    pos6p = jnp.pad(jnp.transpose(posr, (0, 2, 1)), ((0, 0), (0, 2), (0, 0)))
    posp = jnp.pad(posr, ((0, 0), (0, 0), (0, 2)))
    idx4 = idx.reshape(B, M // QB, QB, 1)
    nbr, poss = pl.pallas_call(
        _nbr_kernel,
        grid=(B, M // QB),
        in_specs=[
            pl.BlockSpec((1, 8, NPC), lambda b, q: (b, 0, 0)),
            pl.BlockSpec((1, NPC, 8), lambda b, q: (b, 0, 0)),
            pl.BlockSpec((1, 1, QB, 1), lambda b, q: (b, q, 0, 0)),
        ],
        out_specs=[
            pl.BlockSpec((1, 1, QB, K), lambda b, q: (b, q, 0, 0)),
            pl.BlockSpec((1, 1, QB, 8), lambda b, q: (b, q, 0, 0)),
        ],
        out_shape=[
            jax.ShapeDtypeStruct((B, M // QB, QB, K), jnp.int32),
            jax.ShapeDtypeStruct((B, M // QB, QB, 8), jnp.float32),
        ],
    )(pos6p, posp, idx4)
    return nbr.reshape(B, M, K), poss.reshape(B, M, 8)


def _grouped_conv1d(t, w, b):
    wr = w.reshape(K, K, K)
    return jnp.einsum('tik,imk->tim', t, wr).reshape(t.shape[0], K * K) + b


GCOLS = 144
NW = 32
GCH = 128


def _sc_gather(comb, idxf):
    nrows = idxf.shape[0]
    rpw = nrows // NW
    nch = rpw // GCH
    mesh = plsc.VectorSubcoreMesh(core_axis_name="c", subcore_axis_name="s")

    @functools.partial(
        pl.kernel, mesh=mesh,
        compiler_params=pltpu.CompilerParams(use_tc_tiling_on_sc=False),
        out_type=jax.ShapeDtypeStruct((nrows, GCOLS), jnp.float32),
        scratch_types=[
            pltpu.VMEM((GCH,), jnp.int32),
            pltpu.VMEM((GCH, GCOLS), jnp.float32),
            pltpu.SemaphoreType.DMA,
        ],
    )
    def k(comb_hbm, idx_hbm, out_hbm, idx_v, rows_v, sem):
        wid = lax.axis_index("s") * 2 + lax.axis_index("c")
        base = wid * rpw

        def body(i, carry):
            off = base + i * GCH
            pltpu.sync_copy(idx_hbm.at[pl.ds(off, GCH)], idx_v)
            pltpu.async_copy(comb_hbm.at[idx_v], rows_v, sem).wait()
            pltpu.sync_copy(rows_v, out_hbm.at[pl.ds(off, GCH)])
            return carry

        lax.fori_loop(0, nch, body, 0)

    return k(comb, idxf)


QD = 256


def _dense_kernel(g_ref, ps_ref, w1_ref, b1_ref, w2_ref, b2_ref, w3e_ref,
                  b3_ref, wbd1_ref, cb1_ref, wbd2_ref, cb2_ref, dwt_ref,
                  db_ref, wl_ref, bl_ref, out_ref):
    R = QD * K

    def dot(a, b):
        return jax.lax.dot_general(a, b, (((1,), (0,)), ((), ())),
                                   preferred_element_type=jnp.float32)

    def elu(v):
        return jnp.where(v > 0, v, jnp.exp(jnp.minimum(v, 0.0)) - 1.0)

    g = g_ref[...]
    gx = g[:, :CIN]
    gp = g[:, CIN:CIN + 8]
    ps = ps_ref[...]
    ps_r = jnp.broadcast_to(ps[:, None, :], (QD, K, 8)).reshape(R, 8)
    pr = gp - ps_r
    h1 = elu(dot(pr, w1_ref[...]) + b1_ref[...])
    delta = elu(dot(h1, w2_ref[...]) + b2_ref[...])
    rowk = jax.lax.broadcasted_iota(jnp.int32, (R, 8), 0) % K
    prb = jnp.concatenate(
        [jnp.where(rowk == k, pr, 0.0) for k in range(K)], axis=1)
    t1g = dot(prb, w3e_ref[...])
    t1 = elu(jnp.sum(t1g.reshape(QD, K, K * K), axis=1) + b3_ref[...])
    t2 = elu(dot(t1, wbd1_ref[...]) + cb1_ref[...])
    trf = dot(t2, wbd2_ref[...]) + cb2_ref[...]
    trf_rep = jnp.broadcast_to(trf[:, None, :], (QD, K, K * K)).reshape(R, K * K)
    lane256 = jax.lax.broadcasted_iota(jnp.int32, (R, K * K), 1)
    row256 = jax.lax.broadcasted_iota(jnp.int32, (R, K * K), 0)
    trb = jnp.where((lane256 // K) == (row256 % K), trf_rep, 0.0)
    wkc = dot(trb, dwt_ref[...])
    pd = delta * wkc[:, :CDELTA]
    px = gx * wkc[:, CDELTA:CMID]
    featd = jnp.sum(pd.reshape(QD, K, CDELTA), axis=1)
    featx = jnp.sum(px.reshape(QD, K, CIN), axis=1)
    bias2 = dot(db_ref[...], wl_ref[...]) + bl_ref[...]
    out_ref[...] = (dot(featd, wl_ref[:CDELTA, :])
                    + dot(featx, wl_ref[CDELTA:, :]) + bias2)


def _dense_pallas(g_rows, poss8, w1, b1, w2, b2, w3, b3, cw1, cb1, cw2, cb2,
                  dw, db, wl, bl):
    w1p = jnp.pad(w1, ((0, 2), (0, 0)))
    w3r = w3.reshape(K, D, K * K)
    w3e = jnp.pad(w3r, ((0, 0), (0, 2), (0, 0))).reshape(128, K * K)
    wr1 = cw1.reshape(K, K, K)
    wr2 = cw2.reshape(K, K, K)
    eye = jnp.eye(K, dtype=jnp.float32)
    wbd1 = jnp.einsum('imk,ij->ikjm', wr1, eye).reshape(K * K, K * K)
    wbd2 = jnp.einsum('imk,ij->ikjm', wr2, eye).reshape(K * K, K * K)
    dwt = jnp.tile(jnp.transpose(dw), (K, 1))
    grid = (T // QD,)
    full = lambda r, c: pl.BlockSpec((r, c), lambda i: (0, 0))
    return pl.pallas_call(
        _dense_kernel,
        grid=grid,
        in_specs=[
            pl.BlockSpec((QD * K, GCOLS), lambda i: (i, 0)),
            pl.BlockSpec((QD, 8), lambda i: (i, 0)),
            full(8, CDELTA), full(1, CDELTA),
            full(CDELTA, CDELTA), full(1, CDELTA),
            full(128, K * K), full(1, K * K),
            full(K * K, K * K), full(1, K * K),
            full(K * K, K * K), full(1, K * K),
            full(K * K, CMID),
            full(1, CMID), full(CMID, COUT), full(1, COUT),
        ],
        out_specs=pl.BlockSpec((QD, COUT), lambda i: (i, 0)),
        out_shape=jax.ShapeDtypeStruct((T, COUT), jnp.float32),
    )(g_rows, poss8, w1p, b1.reshape(1, -1), w2, b2.reshape(1, -1),
      w3e, b3.reshape(1, -1), wbd1, cb1.reshape(1, -1), wbd2,
      cb2.reshape(1, -1), dwt, db.reshape(1, -1), wl, bl.reshape(1, -1))


def kernel(x, pos, batch, w1, b1, w2, b2, w3, b3, cw1, cb1, cw2, cb2, dw, db, wl, bl):
    posr = pos.reshape(B, NPC, D)
    xr = x.reshape(B, NPC, CIN)
    idx = _fps_pallas(posr)
    nbr16, poss8 = _nbr_pallas(posr, idx)
    comb = jnp.concatenate(
        [x, pos, jnp.zeros((B * NPC, GCOLS - CIN - D), jnp.float32)], axis=1)
    nbrg = (nbr16 + (jnp.arange(B, dtype=jnp.int32) * NPC)[:, None, None])
    g_rows = _sc_gather(comb, nbrg.reshape(T * K))
    out = _dense_pallas(g_rows, poss8.reshape(T, 8), w1, b1, w2, b2, w3, b3,
                        cw1, cb1, cw2, cb2, dw, db, wl, bl)
    pos_out = poss8[..., :D].reshape(T, D)
    batch_out = jnp.repeat(jnp.arange(B, dtype=batch.dtype), M)
    return out, pos_out, batch_out
